# baseline (speedup 1.0000x reference)
import functools

import jax
import jax.numpy as jnp
from jax import lax
from jax.experimental import pallas as pl
from jax.experimental.pallas import tpu as pltpu
from jax.experimental.pallas import tpu_sc as plsc

N = 1048576
NC = 2
NS = 16
L = 16
NW = NC * NS
PER_W = N // NW
C = 2048
NCHUNK = PER_W // C

_mesh = plsc.VectorSubcoreMesh(core_axis_name="c", subcore_axis_name="s")


@functools.partial(
    pl.kernel,
    mesh=_mesh,
    out_type=jax.ShapeDtypeStruct((NW, L), jnp.float32),
    scratch_types=[
        pltpu.VMEM((C,), jnp.int32),
        pltpu.VMEM((C,), jnp.float32),
        pltpu.VMEM((C,), jnp.float32),
        pltpu.VMEM((C,), jnp.float32),
        pltpu.VMEM((C,), jnp.float32),
        pltpu.VMEM((C,), jnp.float32),
        pltpu.VMEM((C,), jnp.float32),
        pltpu.VMEM((C,), jnp.float32),
        pltpu.VMEM((C,), jnp.float32),
        pltpu.VMEM((C,), jnp.float32),
        pltpu.VMEM((L,), jnp.float32),
        pltpu.SemaphoreType.DMA,
    ],
)
def _rmse_partials(y_hbm, j_hbm, k1_hbm, k2_hbm, sub_hbm,
                   a_hbm, u_hbm, lam_hbm, g1_hbm, g2_hbm,
                   out_hbm,
                   idx_v, y_v, j_v, k1_v, k2_v,
                   a_v, u_v, lam_v, g1_v, g2_v,
                   acc_v, sem):
    wid = lax.axis_index("s") * NC + lax.axis_index("c")

    def chunk_body(c, acc):
        pltpu.sync_copy(sub_hbm.at[wid, c], idx_v)
        pltpu.sync_copy(y_hbm.at[wid, c], y_v)
        pltpu.sync_copy(j_hbm.at[wid, c], j_v)
        pltpu.sync_copy(k1_hbm.at[wid, c], k1_v)
        pltpu.sync_copy(k2_hbm.at[wid, c], k2_v)
        pltpu.async_copy(a_hbm.at[idx_v], a_v, sem)
        pltpu.async_copy(u_hbm.at[idx_v], u_v, sem)
        pltpu.async_copy(lam_hbm.at[idx_v], lam_v, sem)
        pltpu.async_copy(g1_hbm.at[idx_v], g1_v, sem)
        pltpu.async_copy(g2_hbm.at[idx_v], g2_v, sem)
        pltpu.make_async_copy(a_hbm.at[idx_v], a_v, sem).wait()
        pltpu.make_async_copy(u_hbm.at[idx_v], u_v, sem).wait()
        pltpu.make_async_copy(lam_hbm.at[idx_v], lam_v, sem).wait()
        pltpu.make_async_copy(g1_hbm.at[idx_v], g1_v, sem).wait()
        pltpu.make_async_copy(g2_hbm.at[idx_v], g2_v, sem).wait()

        def vec_body(v, acc_v16):
            s = pl.ds(v * L, L)
            a = jnp.maximum(a_v[s], 0.0)
            u = jnp.maximum(u_v[s], 0.0)
            lam = 0.2 / (1.0 + jnp.exp(-lam_v[s]))
            g1 = 1.0 / (1.0 + jnp.exp(-g1_v[s]))
            g2 = 1.0 / (1.0 + jnp.exp(-g2_v[s]))
            t = j_v[s] + g1 * k1_v[s] + g2 * k2_v[s]
            mu = a - u * jnp.exp(-lam * t)
            resid = y_v[s] - mu
            return acc_v16 + resid * resid

        return lax.fori_loop(0, C // L, vec_body, acc)

    acc = lax.fori_loop(0, NCHUNK, chunk_body, jnp.zeros((L,), jnp.float32))
    acc_v[...] = acc
    pltpu.sync_copy(acc_v, out_hbm.at[wid])


def kernel(y, j, k1, k2, sub, A, U, Lambda, Gamma1, Gamma2):
    shp = (NW, NCHUNK, C)
    partials = _rmse_partials(
        y.reshape(shp), j.reshape(shp), k1.reshape(shp), k2.reshape(shp),
        sub.astype(jnp.int32).reshape(shp),
        A, U, Lambda, Gamma1, Gamma2,
    )
    return jnp.sqrt(jnp.sum(partials) / N)
